# baseline (speedup 1.0000x reference)
import jax
import jax.numpy as jnp
from jax.experimental import pallas as pl
from jax.experimental.pallas import tpu as pltpu

_E = 8
_ALPHA = 1.702
_BETA = 1.0
_LIMIT = 7.0
_KC = 2


def _moe_kernel(x_ref, lg_ref, w1_ref, bg_ref, bu_ref, w2_ref, b2_ref,
                out_ref, xbf_ref, act_ref, c_ref):
    e = pl.program_id(0)
    k = pl.program_id(1)
    t, h = x_ref.shape
    i_dim = act_ref.shape[0] * act_ref.shape[2]
    iq = i_dim // _KC
    hc = h // _KC
    dn = (((1,), (1,)), ((), ()))

    @pl.when((e == 0) & (k == 0))
    def _():
        xbf_ref[...] = x_ref[...].astype(jnp.bfloat16)

    @pl.when(k == 0)
    def _():
        lg = lg_ref[...]
        lanes = jax.lax.broadcasted_iota(jnp.int32, lg.shape, 1)
        v0 = jnp.max(lg, axis=1, keepdims=True)
        a0 = jnp.min(jnp.where(lg == v0, lanes, _E), axis=1, keepdims=True)
        masked = jnp.where(lanes == a0, -jnp.inf, lg)
        v1 = jnp.max(masked, axis=1, keepdims=True)
        a1 = jnp.min(jnp.where(masked == v1, lanes, _E), axis=1, keepdims=True)
        g1 = 1.0 / (1.0 + jnp.exp(v0 - v1))
        g0 = 1.0 - g1
        c_ref[...] = jnp.where(a0 == e, g0, jnp.where(a1 == e, g1, 0.0))

    @pl.when(k < _KC)
    def _():
        x = xbf_ref[...]
        w1 = w1_ref[0]
        wg = w1[:, :h].astype(jnp.bfloat16)
        wu = w1[:, h:].astype(jnp.bfloat16)
        gate = jax.lax.dot_general(x, wg, dn, preferred_element_type=jnp.float32)
        up = jax.lax.dot_general(x, wu, dn, preferred_element_type=jnp.float32)
        cols = pl.ds(k * iq, iq)
        gate = jnp.minimum(gate + bg_ref[0, 0, cols][None, :], _LIMIT)
        up = jnp.clip(up + bu_ref[0, 0, cols][None, :], -_LIMIT, _LIMIT)
        gate = gate.astype(jnp.bfloat16)
        up = up.astype(jnp.bfloat16)
        act_ref[k] = (gate * jax.nn.sigmoid(jnp.bfloat16(_ALPHA) * gate)
                      * (up + jnp.bfloat16(_BETA)))

    @pl.when(k >= _KC)
    def _():
        r = k - _KC
        w2 = w2_ref[0].astype(jnp.bfloat16)
        y = None
        for q in range(_KC):
            part = jax.lax.dot_general(
                act_ref[q], w2[:, q * iq:(q + 1) * iq], dn,
                preferred_element_type=jnp.float32)
            y = part if y is None else y + part
        cols = pl.ds(r * hc, hc)
        contrib = (y + b2_ref[0, 0, cols][None, :]) * c_ref[...]

        @pl.when(e == 0)
        def _():
            out_ref[:, cols] = contrib

        @pl.when(e > 0)
        def _():
            out_ref[:, cols] = out_ref[:, cols] + contrib


def kernel(hidden_states, expert_logits, gemm1_weights, gemm1_bias,
           gemm2_weights, gemm2_bias):
    t, h = hidden_states.shape
    i_dim = gemm2_weights.shape[2]
    iq = i_dim // _KC
    hc = h // _KC

    w1_view = gemm1_weights.reshape(_E, i_dim, 2 * h)
    bg = gemm1_bias.reshape(_E, i_dim, 2)[..., 0].reshape(_E, 1, i_dim)
    bu = gemm1_bias.reshape(_E, i_dim, 2)[..., 1].reshape(_E, 1, i_dim)
    b2 = gemm2_bias.reshape(_E, 1, h)

    out = pl.pallas_call(
        _moe_kernel,
        grid=(_E, 2 * _KC),
        in_specs=[
            pl.BlockSpec((t, h), lambda e, k: (0, 0)),
            pl.BlockSpec((t, _E), lambda e, k: (0, 0)),
            pl.BlockSpec((1, iq, 2 * h),
                         lambda e, k: (e, jnp.minimum(k, _KC - 1), 0)),
            pl.BlockSpec((1, 1, i_dim), lambda e, k: (e, 0, 0)),
            pl.BlockSpec((1, 1, i_dim), lambda e, k: (e, 0, 0)),
            pl.BlockSpec((1, hc, i_dim),
                         lambda e, k: (e, jnp.maximum(k - _KC, 0), 0)),
            pl.BlockSpec((1, 1, h), lambda e, k: (e, 0, 0)),
        ],
        out_specs=pl.BlockSpec((t, h), lambda e, k: (0, 0)),
        out_shape=jax.ShapeDtypeStruct((t, h), jnp.float32),
        scratch_shapes=[
            pltpu.VMEM((t, h), jnp.bfloat16),
            pltpu.VMEM((_KC, t, iq), jnp.bfloat16),
            pltpu.VMEM((t, 1), jnp.float32),
        ],
        compiler_params=pltpu.CompilerParams(
            dimension_semantics=("arbitrary", "arbitrary")),
    )(hidden_states, expert_logits, w1_view, bg, bu, gemm2_weights, b2)
    return out.astype(hidden_states.dtype)


# speedup vs baseline: 1.0220x; 1.0220x over previous
import jax
import jax.numpy as jnp
from jax.experimental import pallas as pl
from jax.experimental.pallas import tpu as pltpu

_E = 8
_ALPHA = 1.702
_BETA = 1.0
_LIMIT = 7.0
_KC = 2


def _moe_kernel(x_ref, lg_ref, w1_hbm, bg_ref, bu_ref, w2_hbm, b2_ref,
                out_ref, xbf_ref, act_ref, c_ref, w1b_ref, w2b_ref,
                s1_ref, s2_ref):
    e = pl.program_id(0)
    k = pl.program_id(1)
    nk = 2 * _KC
    s = e * nk + k
    t, h = x_ref.shape
    i_dim = act_ref.shape[0] * act_ref.shape[2]
    iq = i_dim // _KC
    hc = h // _KC
    dn = (((1,), (1,)), ((), ()))

    def w1_copy(ee, kk):
        return pltpu.make_async_copy(
            w1_hbm.at[ee, pl.ds(kk * iq, iq), :],
            w1b_ref.at[kk % 2],
            s1_ref.at[kk % 2])

    def w2_copy(ee, rr):
        return pltpu.make_async_copy(
            w2_hbm.at[ee, pl.ds(rr * hc, hc), :],
            w2b_ref.at[rr % 2],
            s2_ref.at[rr % 2])

    @pl.when(s == 0)
    def _():
        w1_copy(0, 0).start()
        w1_copy(0, 1).start()

    s2 = s + 2
    e2 = s2 // nk
    k2 = s2 % nk

    @pl.when((s2 < _E * nk) & (k2 < _KC))
    def _():
        w1_copy(e2, k2).start()

    @pl.when((s2 < _E * nk) & (k2 >= _KC))
    def _():
        w2_copy(e2, k2 - _KC).start()

    @pl.when((e == 0) & (k == 0))
    def _():
        xbf_ref[...] = x_ref[...].astype(jnp.bfloat16)

    @pl.when(k == 0)
    def _():
        lg = lg_ref[...]
        lanes = jax.lax.broadcasted_iota(jnp.int32, lg.shape, 1)
        v0 = jnp.max(lg, axis=1, keepdims=True)
        a0 = jnp.min(jnp.where(lg == v0, lanes, _E), axis=1, keepdims=True)
        masked = jnp.where(lanes == a0, -jnp.inf, lg)
        v1 = jnp.max(masked, axis=1, keepdims=True)
        a1 = jnp.min(jnp.where(masked == v1, lanes, _E), axis=1, keepdims=True)
        g1 = 1.0 / (1.0 + jnp.exp(v0 - v1))
        g0 = 1.0 - g1
        c_ref[...] = jnp.where(a0 == e, g0, jnp.where(a1 == e, g1, 0.0))

    @pl.when(k < _KC)
    def _():
        w1_copy(e, k).wait()
        x = xbf_ref[...]
        w1 = w1b_ref[k % 2]
        wg = w1[:, :h].astype(jnp.bfloat16)
        wu = w1[:, h:].astype(jnp.bfloat16)
        gate = jax.lax.dot_general(x, wg, dn, preferred_element_type=jnp.float32)
        up = jax.lax.dot_general(x, wu, dn, preferred_element_type=jnp.float32)
        cols = pl.ds(k * iq, iq)
        gate = gate + bg_ref[0, 0, cols][None, :]
        up = up + bu_ref[0, 0, cols][None, :]
        gate = jnp.minimum(gate, _LIMIT)
        up = jnp.clip(up, -_LIMIT, _LIMIT)
        act_ref[k] = (gate * jax.nn.sigmoid(_ALPHA * gate)
                      * (up + _BETA)).astype(jnp.bfloat16)

    @pl.when(k >= _KC)
    def _():
        r = k - _KC
        w2_copy(e, r).wait()
        w2 = w2b_ref[r % 2].astype(jnp.bfloat16)
        y = None
        for q in range(_KC):
            part = jax.lax.dot_general(
                act_ref[q], w2[:, q * iq:(q + 1) * iq], dn,
                preferred_element_type=jnp.float32)
            y = part if y is None else y + part
        cols = pl.ds(r * hc, hc)
        contrib = (y + b2_ref[0, 0, cols][None, :]) * c_ref[...]

        @pl.when(e == 0)
        def _():
            out_ref[:, cols] = contrib

        @pl.when(e > 0)
        def _():
            out_ref[:, cols] = out_ref[:, cols] + contrib


def kernel(hidden_states, expert_logits, gemm1_weights, gemm1_bias,
           gemm2_weights, gemm2_bias):
    t, h = hidden_states.shape
    i_dim = gemm2_weights.shape[2]
    iq = i_dim // _KC
    hc = h // _KC

    w1_view = gemm1_weights.reshape(_E, i_dim, 2 * h)
    bg = gemm1_bias.reshape(_E, i_dim, 2)[..., 0].reshape(_E, 1, i_dim)
    bu = gemm1_bias.reshape(_E, i_dim, 2)[..., 1].reshape(_E, 1, i_dim)
    b2 = gemm2_bias.reshape(_E, 1, h)

    out = pl.pallas_call(
        _moe_kernel,
        grid=(_E, 2 * _KC),
        in_specs=[
            pl.BlockSpec((t, h), lambda e, k: (0, 0)),
            pl.BlockSpec((t, _E), lambda e, k: (0, 0)),
            pl.BlockSpec(memory_space=pl.ANY),
            pl.BlockSpec((1, 1, i_dim), lambda e, k: (e, 0, 0)),
            pl.BlockSpec((1, 1, i_dim), lambda e, k: (e, 0, 0)),
            pl.BlockSpec(memory_space=pl.ANY),
            pl.BlockSpec((1, 1, h), lambda e, k: (e, 0, 0)),
        ],
        out_specs=pl.BlockSpec((t, h), lambda e, k: (0, 0)),
        out_shape=jax.ShapeDtypeStruct((t, h), jnp.float32),
        scratch_shapes=[
            pltpu.VMEM((t, h), jnp.bfloat16),
            pltpu.VMEM((_KC, t, iq), jnp.bfloat16),
            pltpu.VMEM((t, 1), jnp.float32),
            pltpu.VMEM((2, iq, 2 * h), jnp.float32),
            pltpu.VMEM((2, hc, i_dim), jnp.float32),
            pltpu.SemaphoreType.DMA((2,)),
            pltpu.SemaphoreType.DMA((2,)),
        ],
        compiler_params=pltpu.CompilerParams(
            dimension_semantics=("arbitrary", "arbitrary")),
    )(hidden_states, expert_logits, w1_view, bg, bu, gemm2_weights, b2)
    return out.astype(hidden_states.dtype)


# speedup vs baseline: 1.0224x; 1.0004x over previous
import jax
import jax.numpy as jnp
from jax.experimental import pallas as pl
from jax.experimental.pallas import tpu as pltpu

_E = 8
_ALPHA = 1.702
_BETA = 1.0
_LIMIT = 7.0
_KC = 2


def _moe_kernel(x_ref, lg_ref, w1_hbm, bg_ref, bu_ref, w2_hbm, b2_ref,
                out_ref, xbf_ref, act_ref, c_ref, w1b_ref, w2b_ref,
                s1_ref, s2_ref):
    e = pl.program_id(0)
    k = pl.program_id(1)
    nk = 2 * _KC
    s = e * nk + k
    t, h = x_ref.shape
    i_dim = act_ref.shape[0] * act_ref.shape[2]
    iq = i_dim // _KC
    hc = h // _KC
    dn = (((1,), (1,)), ((), ()))

    def w1_copy(ee, kk):
        return pltpu.make_async_copy(
            w1_hbm.at[ee, pl.ds(kk * iq, iq), :],
            w1b_ref.at[kk % 2],
            s1_ref.at[kk % 2])

    def w2_copy(ee, rr):
        return pltpu.make_async_copy(
            w2_hbm.at[ee, pl.ds(rr * hc, hc), :],
            w2b_ref.at[rr % 2],
            s2_ref.at[rr % 2])

    @pl.when(s == 0)
    def _():
        w1_copy(0, 0).start()
        w1_copy(0, 1).start()
        w2_copy(0, 0).start()

    s2 = s + 3
    e2 = s2 // nk
    k2 = s2 % nk

    @pl.when((s2 < _E * nk) & (k2 < _KC))
    def _():
        w1_copy(e2, k2).start()

    @pl.when((s2 < _E * nk) & (k2 >= _KC))
    def _():
        w2_copy(e2, k2 - _KC).start()

    @pl.when((e == 0) & (k == 0))
    def _():
        xbf_ref[...] = x_ref[...].astype(jnp.bfloat16)

    @pl.when(k == 0)
    def _():
        lg = lg_ref[...]
        lanes = jax.lax.broadcasted_iota(jnp.int32, lg.shape, 1)
        v0 = jnp.max(lg, axis=1, keepdims=True)
        a0 = jnp.min(jnp.where(lg == v0, lanes, _E), axis=1, keepdims=True)
        masked = jnp.where(lanes == a0, -jnp.inf, lg)
        v1 = jnp.max(masked, axis=1, keepdims=True)
        a1 = jnp.min(jnp.where(masked == v1, lanes, _E), axis=1, keepdims=True)
        g1 = 1.0 / (1.0 + jnp.exp(v0 - v1))
        g0 = 1.0 - g1
        c_ref[...] = jnp.where(a0 == e, g0, jnp.where(a1 == e, g1, 0.0))

    @pl.when(k < _KC)
    def _():
        w1_copy(e, k).wait()
        x = xbf_ref[...]
        w1 = w1b_ref[k % 2]
        wg = w1[:, :h].astype(jnp.bfloat16)
        wu = w1[:, h:].astype(jnp.bfloat16)
        gate = jax.lax.dot_general(x, wg, dn, preferred_element_type=jnp.float32)
        up = jax.lax.dot_general(x, wu, dn, preferred_element_type=jnp.float32)
        cols = pl.ds(k * iq, iq)
        gate = gate + bg_ref[0, 0, cols][None, :]
        up = up + bu_ref[0, 0, cols][None, :]
        gate = jnp.minimum(gate, _LIMIT)
        up = jnp.clip(up, -_LIMIT, _LIMIT)
        act_ref[k] = (gate * jax.nn.sigmoid(_ALPHA * gate)
                      * (up + _BETA)).astype(jnp.bfloat16)

    @pl.when(k >= _KC)
    def _():
        r = k - _KC
        w2_copy(e, r).wait()
        w2 = w2b_ref[r % 2].astype(jnp.bfloat16)
        y = None
        for q in range(_KC):
            part = jax.lax.dot_general(
                act_ref[q], w2[:, q * iq:(q + 1) * iq], dn,
                preferred_element_type=jnp.float32)
            y = part if y is None else y + part
        cols = pl.ds(r * hc, hc)
        contrib = (y + b2_ref[0, 0, cols][None, :]) * c_ref[...]

        @pl.when(e == 0)
        def _():
            out_ref[:, cols] = contrib

        @pl.when(e > 0)
        def _():
            out_ref[:, cols] = out_ref[:, cols] + contrib


def kernel(hidden_states, expert_logits, gemm1_weights, gemm1_bias,
           gemm2_weights, gemm2_bias):
    t, h = hidden_states.shape
    i_dim = gemm2_weights.shape[2]
    iq = i_dim // _KC
    hc = h // _KC

    w1_view = gemm1_weights.reshape(_E, i_dim, 2 * h)
    bg = gemm1_bias.reshape(_E, i_dim, 2)[..., 0].reshape(_E, 1, i_dim)
    bu = gemm1_bias.reshape(_E, i_dim, 2)[..., 1].reshape(_E, 1, i_dim)
    b2 = gemm2_bias.reshape(_E, 1, h)

    out = pl.pallas_call(
        _moe_kernel,
        grid=(_E, 2 * _KC),
        in_specs=[
            pl.BlockSpec((t, h), lambda e, k: (0, 0)),
            pl.BlockSpec((t, _E), lambda e, k: (0, 0)),
            pl.BlockSpec(memory_space=pl.ANY),
            pl.BlockSpec((1, 1, i_dim), lambda e, k: (e, 0, 0)),
            pl.BlockSpec((1, 1, i_dim), lambda e, k: (e, 0, 0)),
            pl.BlockSpec(memory_space=pl.ANY),
            pl.BlockSpec((1, 1, h), lambda e, k: (e, 0, 0)),
        ],
        out_specs=pl.BlockSpec((t, h), lambda e, k: (0, 0)),
        out_shape=jax.ShapeDtypeStruct((t, h), jnp.float32),
        scratch_shapes=[
            pltpu.VMEM((t, h), jnp.bfloat16),
            pltpu.VMEM((_KC, t, iq), jnp.bfloat16),
            pltpu.VMEM((t, 1), jnp.float32),
            pltpu.VMEM((2, iq, 2 * h), jnp.float32),
            pltpu.VMEM((2, hc, i_dim), jnp.float32),
            pltpu.SemaphoreType.DMA((2,)),
            pltpu.SemaphoreType.DMA((2,)),
        ],
        compiler_params=pltpu.CompilerParams(
            dimension_semantics=("arbitrary", "arbitrary")),
    )(hidden_states, expert_logits, w1_view, bg, bu, gemm2_weights, b2)
    return out.astype(hidden_states.dtype)
